# baseline (device time: 149180 ns/iter reference)
import jax
import jax.numpy as jnp
from jax import lax
from jax.experimental import pallas as pl
from jax.experimental.pallas import tpu as pltpu

N_DEV = 16


def kernel(A, B):
    m, k = A.shape
    _, n = B.shape
    chunk = m // N_DEV

    def body(a_ref, b_ref, out_ref, accum_ref, comm_ref, send_sem, recv_sems):
        my = lax.axis_index("i")
        left = (my - 1) % N_DEV
        right = (my + 1) % N_DEV

        barrier = pltpu.get_barrier_semaphore()
        for nbr in (left, right):
            pl.semaphore_signal(
                barrier, inc=1, device_id=(nbr,),
                device_id_type=pl.DeviceIdType.MESH,
            )
        pl.semaphore_wait(barrier, 2)

        a = a_ref[...].astype(jnp.bfloat16)
        b = b_ref[...].astype(jnp.bfloat16)
        accum_ref[...] = jnp.dot(a, b, preferred_element_type=jnp.float32)

        for s in range(N_DEV - 1):
            send_idx = (my - s) % N_DEV
            rdma = pltpu.make_async_remote_copy(
                src_ref=accum_ref.at[pl.ds(send_idx * chunk, chunk), :],
                dst_ref=comm_ref.at[s],
                send_sem=send_sem,
                recv_sem=recv_sems.at[s],
                device_id=(right,),
                device_id_type=pl.DeviceIdType.MESH,
            )
            rdma.start()
            rdma.wait()
            recv_idx = (my - s - 1) % N_DEV
            sl = pl.ds(recv_idx * chunk, chunk)
            accum_ref[sl, :] = accum_ref[sl, :] + comm_ref[s]

        own = (my + 1) % N_DEV
        own_sl = pl.ds(own * chunk, chunk)
        out_ref[own_sl, :] = jnp.maximum(accum_ref[own_sl, :], 0.0)

        for t in range(N_DEV - 1):
            g = (my + 1 - t) % N_DEV
            gsl = pl.ds(g * chunk, chunk)
            rdma = pltpu.make_async_remote_copy(
                src_ref=out_ref.at[gsl, :],
                dst_ref=out_ref.at[gsl, :],
                send_sem=send_sem,
                recv_sem=recv_sems.at[N_DEV - 1 + t],
                device_id=(right,),
                device_id_type=pl.DeviceIdType.MESH,
            )
            rdma.start()
            rdma.wait()

    return pl.pallas_call(
        body,
        out_shape=jax.ShapeDtypeStruct((m, n), jnp.float32),
        in_specs=[
            pl.BlockSpec(memory_space=pltpu.VMEM),
            pl.BlockSpec(memory_space=pltpu.VMEM),
        ],
        out_specs=pl.BlockSpec(memory_space=pltpu.VMEM),
        scratch_shapes=[
            pltpu.VMEM((m, n), jnp.float32),
            pltpu.VMEM((N_DEV - 1, chunk, n), jnp.float32),
            pltpu.SemaphoreType.DMA,
            pltpu.SemaphoreType.DMA((2 * (N_DEV - 1),)),
        ],
        compiler_params=pltpu.CompilerParams(collective_id=0),
    )(A, B)


# device time: 67821 ns/iter; 2.1996x vs baseline; 2.1996x over previous
import jax
import jax.numpy as jnp
from jax import lax
from jax.experimental import pallas as pl
from jax.experimental.pallas import tpu as pltpu

N_DEV = 16


def kernel(A, B):
    m, k = A.shape
    _, n = B.shape

    h1, h2, h3, h4 = m // 2, m // 4, m // 8, m // 16

    def body(a_ref, b_ref, out_ref, accum_ref, stage_ref, recv_ref,
             send_sem, rs_sems, ag_sems):
        my = lax.axis_index("i")
        q0 = my & 1
        q1 = (my >> 1) & 1
        hx = q0 ^ q1
        hy = q1
        z0 = (my >> 2) & 1
        z1 = (my >> 3) & 1

        partners = [my ^ 1, my ^ 3, my ^ 4, my ^ 8]

        barrier = pltpu.get_barrier_semaphore()
        for p in partners:
            pl.semaphore_signal(
                barrier, inc=1, device_id=(p,),
                device_id_type=pl.DeviceIdType.MESH,
            )
        pl.semaphore_wait(barrier, len(partners))

        a = a_ref[...].astype(jnp.bfloat16)
        b = b_ref[...].astype(jnp.bfloat16)
        accum_ref[...] = jnp.dot(a, b, preferred_element_type=jnp.float32)

        rs_steps = [
            (partners[0], h1, (1 - hx) * h1, 0),
            (partners[1], h2, hx * h1 + (1 - hy) * h2, h1),
            (partners[2], h3, hx * h1 + hy * h2 + (1 - z0) * h3, h1 + h2),
            (partners[3], h4,
             hx * h1 + hy * h2 + z0 * h3 + (1 - z1) * h4, h1 + h2 + h3),
        ]
        kept_offs = [
            hx * h1,
            hx * h1 + hy * h2,
            hx * h1 + hy * h2 + z0 * h3,
            hx * h1 + hy * h2 + z0 * h3 + z1 * h4,
        ]
        for s, (p, rows, send_off, slot_off) in enumerate(rs_steps):
            stage_ref[pl.ds(0, rows), :] = (
                accum_ref[pl.ds(send_off, rows), :].astype(jnp.bfloat16)
            )
            rdma = pltpu.make_async_remote_copy(
                src_ref=stage_ref.at[pl.ds(0, rows), :],
                dst_ref=recv_ref.at[pl.ds(slot_off, rows), :],
                send_sem=send_sem,
                recv_sem=rs_sems.at[s],
                device_id=(p,),
                device_id_type=pl.DeviceIdType.MESH,
            )
            rdma.start()
            rdma.wait()
            ksl = pl.ds(kept_offs[s], rows)
            accum_ref[ksl, :] = accum_ref[ksl, :] + recv_ref[
                pl.ds(slot_off, rows), :
            ].astype(jnp.float32)

        own_off = kept_offs[3]
        out_ref[pl.ds(own_off, h4), :] = jnp.maximum(
            accum_ref[pl.ds(own_off, h4), :], 0.0
        ).astype(jnp.bfloat16)

        ag_steps = [
            (partners[3], h4, own_off),
            (partners[2], h3, hx * h1 + hy * h2 + z0 * h3),
            (partners[1], h2, hx * h1 + hy * h2),
            (partners[0], h1, hx * h1),
        ]
        for t, (p, rows, blk_off) in enumerate(ag_steps):
            bsl = pl.ds(blk_off, rows)
            rdma = pltpu.make_async_remote_copy(
                src_ref=out_ref.at[bsl, :],
                dst_ref=out_ref.at[bsl, :],
                send_sem=send_sem,
                recv_sem=ag_sems.at[t],
                device_id=(p,),
                device_id_type=pl.DeviceIdType.MESH,
            )
            rdma.start()
            rdma.wait()

    return pl.pallas_call(
        body,
        out_shape=jax.ShapeDtypeStruct((m, n), jnp.bfloat16),
        in_specs=[
            pl.BlockSpec(memory_space=pltpu.VMEM),
            pl.BlockSpec(memory_space=pltpu.VMEM),
        ],
        out_specs=pl.BlockSpec(memory_space=pltpu.VMEM),
        scratch_shapes=[
            pltpu.VMEM((m, n), jnp.float32),
            pltpu.VMEM((h1, n), jnp.bfloat16),
            pltpu.VMEM((h1 + h2 + h3 + h4, n), jnp.bfloat16),
            pltpu.SemaphoreType.DMA,
            pltpu.SemaphoreType.DMA((4,)),
            pltpu.SemaphoreType.DMA((4,)),
        ],
        compiler_params=pltpu.CompilerParams(collective_id=0),
    )(A, B)


# device time: 51854 ns/iter; 2.8769x vs baseline; 1.3079x over previous
import jax
import jax.numpy as jnp
from jax import lax
from jax.experimental import pallas as pl
from jax.experimental.pallas import tpu as pltpu

N_DEV = 16


def kernel(A, B):
    m, k = A.shape
    _, n = B.shape

    h1, h2, h3, h4 = m // 2, m // 4, m // 8, m // 16

    def body(a_ref, b_ref, out_ref, accum_ref, stage_ref, recv_ref,
             send_sems, rs_sems, ag_sems):
        my = lax.axis_index("i")
        q0 = my & 1
        q1 = (my >> 1) & 1
        hx = q0 ^ q1
        hy = q1
        z0 = (my >> 2) & 1
        z1 = (my >> 3) & 1

        partners = [my ^ 1, my ^ 3, my ^ 4, my ^ 8]

        barrier = pltpu.get_barrier_semaphore()
        for p in partners:
            pl.semaphore_signal(
                barrier, inc=1, device_id=(p,),
                device_id_type=pl.DeviceIdType.MESH,
            )
        pl.semaphore_wait(barrier, len(partners))

        a = a_ref[...].astype(jnp.bfloat16)
        b = b_ref[...].astype(jnp.bfloat16)
        accum_ref[...] = jnp.dot(a, b, preferred_element_type=jnp.float32)

        nh = n // 2
        def rs_sched(first_x):
            if first_x:
                b1, o1, b2, o2 = hx, (1 - hx) * h1, hy, (1 - hy) * h2
            else:
                b1, o1, b2, o2 = hy, (1 - hy) * h1, hx, (1 - hx) * h2
            p1, p2 = (partners[0], partners[1]) if first_x else (
                partners[1], partners[0])
            k1 = b1 * h1
            k2 = k1 + b2 * h2
            k3 = k2 + z0 * h3
            k4 = k3 + z1 * h4
            return [
                (p1, h1, o1, k1),
                (p2, h2, k1 + o2, k2),
                (partners[2], h3, k2 + (1 - z0) * h3, k3),
                (partners[3], h4, k3 + (1 - z1) * h4, k4),
            ]

        scheds = [rs_sched(True), rs_sched(False)]
        slot_offs = [0, h1, h1 + h2, h1 + h2 + h3]
        for s in range(4):
            rdmas = []
            for st in range(2):
                p, rows, send_off, _ = scheds[st][s]
                csl = pl.ds(st * nh, nh)
                stage_ref[pl.ds(0, rows), csl] = (
                    accum_ref[pl.ds(send_off, rows), csl].astype(jnp.bfloat16)
                )
                rdma = pltpu.make_async_remote_copy(
                    src_ref=stage_ref.at[pl.ds(0, rows), csl],
                    dst_ref=recv_ref.at[pl.ds(slot_offs[s], rows), csl],
                    send_sem=send_sems.at[st],
                    recv_sem=rs_sems.at[2 * s + st],
                    device_id=(p,),
                    device_id_type=pl.DeviceIdType.MESH,
                )
                rdma.start()
                rdmas.append(rdma)
            for st in range(2):
                _, rows, _, kept_off = scheds[st][s]
                rdmas[st].wait()
                csl = pl.ds(st * nh, nh)
                ksl = pl.ds(kept_off, rows)
                accum_ref[ksl, csl] = accum_ref[ksl, csl] + recv_ref[
                    pl.ds(slot_offs[s], rows), csl
                ].astype(jnp.float32)

        for st in range(2):
            own_off = scheds[st][3][3]
            csl = pl.ds(st * nh, nh)
            out_ref[pl.ds(own_off, h4), csl] = jnp.maximum(
                accum_ref[pl.ds(own_off, h4), csl], 0.0
            ).astype(jnp.bfloat16)

        for t in range(4):
            rdmas = []
            for st in range(2):
                s = 3 - t
                p, rows, _, kept = scheds[st][s]
                csl = pl.ds(st * nh, nh)
                bsl = pl.ds(kept, rows)
                rdma = pltpu.make_async_remote_copy(
                    src_ref=out_ref.at[bsl, csl],
                    dst_ref=out_ref.at[bsl, csl],
                    send_sem=send_sems.at[st],
                    recv_sem=ag_sems.at[2 * t + st],
                    device_id=(p,),
                    device_id_type=pl.DeviceIdType.MESH,
                )
                rdma.start()
                rdmas.append(rdma)
            for r in rdmas:
                r.wait()

    return pl.pallas_call(
        body,
        out_shape=jax.ShapeDtypeStruct((m, n), jnp.bfloat16),
        in_specs=[
            pl.BlockSpec(memory_space=pltpu.VMEM),
            pl.BlockSpec(memory_space=pltpu.VMEM),
        ],
        out_specs=pl.BlockSpec(memory_space=pltpu.VMEM),
        scratch_shapes=[
            pltpu.VMEM((m, n), jnp.float32),
            pltpu.VMEM((h1, n), jnp.bfloat16),
            pltpu.VMEM((h1 + h2 + h3 + h4, n), jnp.bfloat16),
            pltpu.SemaphoreType.DMA((2,)),
            pltpu.SemaphoreType.DMA((8,)),
            pltpu.SemaphoreType.DMA((8,)),
        ],
        compiler_params=pltpu.CompilerParams(collective_id=0),
    )(A, B)


# device time: 51138 ns/iter; 2.9172x vs baseline; 1.0140x over previous
import jax
import jax.numpy as jnp
from jax import lax
from jax.experimental import pallas as pl
from jax.experimental.pallas import tpu as pltpu

N_DEV = 16


def kernel(A, B):
    m, k = A.shape
    _, n = B.shape

    h1, h2, h3, h4 = m // 2, m // 4, m // 8, m // 16

    def body(a_ref, b_ref, out_ref, accum_ref, recv_ref,
             send_sems, rs_sems, ag_sems):
        my = lax.axis_index("i")
        q0 = my & 1
        q1 = (my >> 1) & 1
        hx = q0 ^ q1
        hy = q1
        z0 = (my >> 2) & 1
        z1 = (my >> 3) & 1

        partners = [my ^ 1, my ^ 3, my ^ 4, my ^ 8]

        barrier = pltpu.get_barrier_semaphore()
        for p in partners:
            pl.semaphore_signal(
                barrier, inc=1, device_id=(p,),
                device_id_type=pl.DeviceIdType.MESH,
            )
        pl.semaphore_wait(barrier, len(partners))

        nh = n // 2
        b = b_ref[...].astype(jnp.bfloat16)
        def rs_sched(first_x):
            if first_x:
                b1, o1, b2, o2 = hx, (1 - hx) * h1, hy, (1 - hy) * h2
            else:
                b1, o1, b2, o2 = hy, (1 - hy) * h1, hx, (1 - hx) * h2
            p1, p2 = (partners[0], partners[1]) if first_x else (
                partners[1], partners[0])
            k1 = b1 * h1
            k2 = k1 + b2 * h2
            k3 = k2 + z0 * h3
            k4 = k3 + z1 * h4
            return [
                (p1, h1, o1, k1),
                (p2, h2, k1 + o2, k2),
                (partners[2], h3, k2 + (1 - z0) * h3, k3),
                (partners[3], h4, k3 + (1 - z1) * h4, k4),
            ]

        scheds = [rs_sched(True), rs_sched(False)]
        slot_offs = [0, h1, h1 + h2, h1 + h2 + h3]

        def start_rs(st, s):
            p, rows, send_off, _ = scheds[st][s]
            csl = pl.ds(st * nh, nh)
            rdma = pltpu.make_async_remote_copy(
                src_ref=accum_ref.at[pl.ds(send_off, rows), csl],
                dst_ref=recv_ref.at[pl.ds(slot_offs[s], rows), csl],
                send_sem=send_sems.at[st],
                recv_sem=rs_sems.at[2 * s + st],
                device_id=(p,),
                device_id_type=pl.DeviceIdType.MESH,
            )
            rdma.start()
            return rdma

        def finish_rs(st, s, rdma):
            _, rows, _, kept_off = scheds[st][s]
            rdma.wait()
            csl = pl.ds(st * nh, nh)
            ksl = pl.ds(kept_off, rows)
            accum_ref[ksl, csl] = accum_ref[ksl, csl] + recv_ref[
                pl.ds(slot_offs[s], rows), csl
            ]

        for st in range(2):
            _, rows, send_off, _ = scheds[st][0]
            csl = pl.ds(st * nh, nh)
            a_q = a_ref[pl.ds(send_off, rows), :].astype(jnp.bfloat16)
            accum_ref[pl.ds(send_off, rows), csl] = jnp.dot(
                a_q, b[:, st * nh:(st + 1) * nh],
                preferred_element_type=jnp.float32,
            ).astype(jnp.bfloat16)
        rdmas = [start_rs(0, 0), start_rs(1, 0)]
        for st in range(2):
            _, rows, _, kept_off = scheds[st][0]
            csl = pl.ds(st * nh, nh)
            a_q = a_ref[pl.ds(kept_off, rows), :].astype(jnp.bfloat16)
            accum_ref[pl.ds(kept_off, rows), csl] = jnp.dot(
                a_q, b[:, st * nh:(st + 1) * nh],
                preferred_element_type=jnp.float32,
            ).astype(jnp.bfloat16)
        for st in range(2):
            finish_rs(st, 0, rdmas[st])

        for s in range(1, 4):
            rdmas = [start_rs(0, s), start_rs(1, s)]
            for st in range(2):
                finish_rs(st, s, rdmas[st])

        for st in range(2):
            own_off = scheds[st][3][3]
            csl = pl.ds(st * nh, nh)
            out_ref[pl.ds(own_off, h4), csl] = jnp.maximum(
                accum_ref[pl.ds(own_off, h4), csl], 0.0
            ).astype(jnp.bfloat16)

        for t in range(4):
            rdmas = []
            for st in range(2):
                s = 3 - t
                p, rows, _, kept = scheds[st][s]
                csl = pl.ds(st * nh, nh)
                bsl = pl.ds(kept, rows)
                rdma = pltpu.make_async_remote_copy(
                    src_ref=out_ref.at[bsl, csl],
                    dst_ref=out_ref.at[bsl, csl],
                    send_sem=send_sems.at[st],
                    recv_sem=ag_sems.at[2 * t + st],
                    device_id=(p,),
                    device_id_type=pl.DeviceIdType.MESH,
                )
                rdma.start()
                rdmas.append(rdma)
            for r in rdmas:
                r.wait()

    return pl.pallas_call(
        body,
        out_shape=jax.ShapeDtypeStruct((m, n), jnp.bfloat16),
        in_specs=[
            pl.BlockSpec(memory_space=pltpu.VMEM),
            pl.BlockSpec(memory_space=pltpu.VMEM),
        ],
        out_specs=pl.BlockSpec(memory_space=pltpu.VMEM),
        scratch_shapes=[
            pltpu.VMEM((m, n), jnp.bfloat16),
            pltpu.VMEM((h1 + h2 + h3 + h4, n), jnp.bfloat16),
            pltpu.SemaphoreType.DMA((2,)),
            pltpu.SemaphoreType.DMA((8,)),
            pltpu.SemaphoreType.DMA((8,)),
        ],
        compiler_params=pltpu.CompilerParams(collective_id=0),
    )(A, B)


# device time: 42411 ns/iter; 3.5175x vs baseline; 1.2058x over previous
import jax
import jax.numpy as jnp
from jax import lax
from jax.experimental import pallas as pl
from jax.experimental.pallas import tpu as pltpu

N_DEV = 16


def kernel(A, B):
    m, k = A.shape
    _, n = B.shape

    h1, h2, h3, h4 = m // 2, m // 4, m // 8, m // 16
    nh = n // 2
    nq = n // 4

    def body(a_ref, b_ref, out_ref, accum_ref, recv_ref,
             send_sems, rs_sems, ag_sems):
        my = lax.axis_index("i")
        q0 = my & 1
        q1 = (my >> 1) & 1
        hx = q0 ^ q1
        hy = q1
        z0 = (my >> 2) & 1
        z1 = (my >> 3) & 1

        partners = [my ^ 1, my ^ 3, my ^ 4, my ^ 8]

        barrier = pltpu.get_barrier_semaphore()
        for p in partners + [my ^ 12]:
            pl.semaphore_signal(
                barrier, inc=1, device_id=(p,),
                device_id_type=pl.DeviceIdType.MESH,
            )
        pl.semaphore_wait(barrier, 5)

        def rs_sched(first_x):
            if first_x:
                b1, o1, b2, o2 = hx, (1 - hx) * h1, hy, (1 - hy) * h2
            else:
                b1, o1, b2, o2 = hy, (1 - hy) * h1, hx, (1 - hx) * h2
            p1, p2 = (partners[0], partners[1]) if first_x else (
                partners[1], partners[0])
            k1 = b1 * h1
            k2 = k1 + b2 * h2
            k3 = k2 + z0 * h3
            k4 = k3 + z1 * h4
            return [
                (p1, h1, o1, k1),
                (p2, h2, k1 + o2, k2),
                (partners[2], h3, k2 + (1 - z0) * h3, k3),
                (partners[3], h4, k3 + (1 - z1) * h4, k4),
            ]

        scheds = [rs_sched(True), rs_sched(False)]
        slot_offs = [0, h1, h1 + h2, h1 + h2 + h3]

        def cols(st, c):
            if c is None:
                return pl.ds(st * nh, nh)
            return pl.ds(st * nh + c * nq, nq)

        def rs_sem(s, st, c):
            if s < 2:
                return rs_sems.at[s * 4 + st * 2 + c]
            return rs_sems.at[8 + (s - 2) * 2 + st]

        def ag_sem(t, st, c):
            return ag_sems.at[8 + (t - 2) * 4 + st * 2 + c]

        def chain(st, c):
            return send_sems.at[st * 2 + (0 if c is None else c)]

        def start_rs(st, s, c):
            p, rows, send_off, _ = scheds[st][s]
            csl = cols(st, c)
            rdma = pltpu.make_async_remote_copy(
                src_ref=accum_ref.at[pl.ds(send_off, rows), csl],
                dst_ref=recv_ref.at[pl.ds(slot_offs[s], rows), csl],
                send_sem=chain(st, c),
                recv_sem=rs_sem(s, st, 0 if c is None else c),
                device_id=(p,),
                device_id_type=pl.DeviceIdType.MESH,
            )
            rdma.start()
            return rdma

        def finish_rs(st, s, c, rdma):
            _, rows, _, kept_off = scheds[st][s]
            rdma.wait()
            csl = cols(st, c)
            ksl = pl.ds(kept_off, rows)
            accum_ref[ksl, csl] = accum_ref[ksl, csl] + recv_ref[
                pl.ds(slot_offs[s], rows), csl
            ]

        rs1 = {}
        for c in range(2):
            for st in range(2):
                _, rows, send_off, _ = scheds[st][0]
                a_q = a_ref[pl.ds(send_off, rows), :]
                ccol = st * nh + c * nq
                accum_ref[pl.ds(send_off, rows), cols(st, c)] = jnp.dot(
                    a_q, b_ref[:, ccol:ccol + nq],
                    preferred_element_type=jnp.float32,
                ).astype(jnp.bfloat16)
                rs1[(st, c)] = start_rs(st, 0, c)
        for st in range(2):
            _, rows, _, kept_off = scheds[st][0]
            a_q = a_ref[pl.ds(kept_off, rows), :]
            accum_ref[pl.ds(kept_off, rows), cols(st, None)] = jnp.dot(
                a_q, b_ref[:, st * nh:(st + 1) * nh],
                preferred_element_type=jnp.float32,
            ).astype(jnp.bfloat16)

        rs2 = {}
        for c in range(2):
            for st in range(2):
                finish_rs(st, 0, c, rs1[(st, c)])
                rs2[(st, c)] = start_rs(st, 1, c)
        zbase = h1 + h2
        my_z = (my >> 2) & 3

        def zoff(v):
            return (v & 1) * h3 + ((v >> 1) & 1) * h4

        def z_peer_id(zp):
            return (my & 3) + zp * 4

        def zchain(st, j):
            return send_sems.at[4 + st * 3 + j]

        def start_zrs(st, dz):
            zp = my_z ^ dz
            k2 = scheds[st][1][3]
            rdma = pltpu.make_async_remote_copy(
                src_ref=accum_ref.at[pl.ds(k2 + zoff(zp), h4), cols(st, None)],
                dst_ref=recv_ref.at[pl.ds(zbase + my_z * h4, h4),
                                    cols(st, None)],
                send_sem=zchain(st, dz - 1),
                recv_sem=rs_sems.at[8 + st * 4 + my_z],
                device_id=(z_peer_id(zp),),
                device_id_type=pl.DeviceIdType.MESH,
            )
            rdma.start()
            return rdma

        def recv_zrs(st, dz):
            zs = my_z ^ dz
            dst = recv_ref.at[pl.ds(zbase + zs * h4, h4), cols(st, None)]
            rdma = pltpu.make_async_remote_copy(
                src_ref=dst,
                dst_ref=dst,
                send_sem=zchain(st, dz - 1),
                recv_sem=rs_sems.at[8 + st * 4 + zs],
                device_id=(z_peer_id(zs),),
                device_id_type=pl.DeviceIdType.MESH,
            )
            rdma.wait_recv()

        zrs = {}
        for st in range(2):
            for c in range(2):
                finish_rs(st, 1, c, rs2[(st, c)])
            for dz in (1, 2, 3):
                zrs[(st, dz)] = start_zrs(st, dz)

        def start_zag(st, dz):
            zp = my_z ^ dz
            own = scheds[st][3][3]
            src = out_ref.at[pl.ds(own, h4), cols(st, None)]
            rdma = pltpu.make_async_remote_copy(
                src_ref=src,
                dst_ref=src,
                send_sem=zchain(st, dz - 1),
                recv_sem=ag_sems.at[st * 4 + my_z],
                device_id=(z_peer_id(zp),),
                device_id_type=pl.DeviceIdType.MESH,
            )
            rdma.start()
            return rdma

        def recv_zag(st, dz):
            zs = my_z ^ dz
            k2 = scheds[st][1][3]
            dst = out_ref.at[pl.ds(k2 + zoff(zs), h4), cols(st, None)]
            rdma = pltpu.make_async_remote_copy(
                src_ref=dst,
                dst_ref=dst,
                send_sem=zchain(st, dz - 1),
                recv_sem=ag_sems.at[st * 4 + zs],
                device_id=(z_peer_id(zs),),
                device_id_type=pl.DeviceIdType.MESH,
            )
            rdma.wait_recv()

        zag = {}
        for st in range(2):
            own_off = scheds[st][3][3]
            csl = cols(st, None)
            osl = pl.ds(own_off, h4)
            for dz in (1, 2, 3):
                recv_zrs(st, dz)
                zs = my_z ^ dz
                k2 = scheds[st][1][3]
                accum_ref[osl, csl] = accum_ref[osl, csl] + recv_ref[
                    pl.ds(zbase + zs * h4, h4), csl
                ]
            out_ref[osl, csl] = jnp.maximum(
                accum_ref[osl, csl], 0.0
            ).astype(jnp.bfloat16)
            for dz in (1, 2, 3):
                zrs[(st, dz)].wait_send()
                zag[(st, dz)] = start_zag(st, dz)

        def start_ag(st, t, c):
            s = 3 - t
            p, rows, _, kept = scheds[st][s]
            csl = cols(st, c)
            bsl = pl.ds(kept, rows)
            rdma = pltpu.make_async_remote_copy(
                src_ref=out_ref.at[bsl, csl],
                dst_ref=out_ref.at[bsl, csl],
                send_sem=chain(st, c),
                recv_sem=ag_sem(t, st, 0 if c is None else c),
                device_id=(p,),
                device_id_type=pl.DeviceIdType.MESH,
            )
            rdma.start()
            return rdma

        ag3 = {}
        for st in range(2):
            for dz in (1, 2, 3):
                recv_zag(st, dz)
            for c in range(2):
                ag3[(st, c)] = start_ag(st, 2, c)
        ag4 = {}
        for c in range(2):
            for st in range(2):
                ag3[(st, c)].wait()
                ag4[(st, c)] = start_ag(st, 3, c)
        for c in range(2):
            for st in range(2):
                ag4[(st, c)].wait()
        for st in range(2):
            for dz in (1, 2, 3):
                zag[(st, dz)].wait_send()

    return pl.pallas_call(
        body,
        out_shape=jax.ShapeDtypeStruct((m, n), jnp.bfloat16),
        in_specs=[
            pl.BlockSpec(memory_space=pltpu.VMEM),
            pl.BlockSpec(memory_space=pltpu.VMEM),
        ],
        out_specs=pl.BlockSpec(memory_space=pltpu.VMEM),
        scratch_shapes=[
            pltpu.VMEM((m, n), jnp.bfloat16),
            pltpu.VMEM((h1 + h2 + 4 * h4, n), jnp.bfloat16),
            pltpu.SemaphoreType.DMA((10,)),
            pltpu.SemaphoreType.DMA((16,)),
            pltpu.SemaphoreType.DMA((16,)),
        ],
        compiler_params=pltpu.CompilerParams(collective_id=0),
    )(A.astype(jnp.bfloat16), B.astype(jnp.bfloat16))
